# Initial kernel scaffold; baseline (speedup 1.0000x reference)
import jax
import jax.numpy as jnp
from jax.experimental import pallas as pl


def kernel(x_in, table):
    raise NotImplementedError("write your pallas kernel here")


# speedup vs baseline: 2.0862x; 2.0862x over previous
import dataclasses
import functools

import jax
import jax.numpy as jnp
import numpy as np
from jax import lax
from jax.experimental import pallas as pl
from jax.experimental.pallas import tpu as pltpu
from jax.experimental.pallas import tpu_sc as plsc

MAX_POS = 200
EMBED_DIM = 64
LANES = 16

NUM_CORES = 2
NUM_SUBCORES = 16
NUM_WORKERS = NUM_CORES * NUM_SUBCORES

CHUNK = 512
GATHER_W = 128
GATHERS = CHUNK // GATHER_W


def _pos_enc_ext(num_rows: int) -> np.ndarray:
    pos = np.arange(MAX_POS)[:, np.newaxis]
    i = np.arange(EMBED_DIM)[np.newaxis, :]
    angle_rates = 1 / np.power(10000, 2 * (i // 2) / np.float32(EMBED_DIM))
    angle_rads = pos * angle_rates
    angle_rads[:, 0::2] = np.sin(angle_rads[:, 0::2])
    angle_rads[:, 1::2] = np.cos(angle_rads[:, 1::2])
    pe = angle_rads.astype(np.float32)
    return pe[np.arange(num_rows) % MAX_POS]


def _make_sc_embed(n_tokens: int):
    tok_per_w = n_tokens // NUM_WORKERS
    n_chunks = tok_per_w // CHUNK
    ext_rows = 192 + CHUNK
    mesh = plsc.VectorSubcoreMesh(core_axis_name="c", subcore_axis_name="s")
    cp = pltpu.CompilerParams()
    if "needs_layout_passes" in pltpu.CompilerParams.__dataclass_fields__:
        cp = dataclasses.replace(cp, needs_layout_passes=False)
    if "use_tc_tiling_on_sc" in pltpu.CompilerParams.__dataclass_fields__:
        cp = dataclasses.replace(cp, use_tc_tiling_on_sc=False)

    @functools.partial(
        pl.kernel,
        compiler_params=cp,
        out_type=jax.ShapeDtypeStruct((n_tokens, EMBED_DIM), jnp.float32),
        mesh=mesh,
        scratch_types=[
            pltpu.VMEM((GATHERS, GATHER_W), jnp.int32),
            pltpu.VMEM((CHUNK,), jnp.float32),
            pltpu.VMEM((CHUNK, EMBED_DIM), jnp.float32),
            pltpu.VMEM((ext_rows, EMBED_DIM), jnp.float32),
            pltpu.SemaphoreType.DMA,
        ],
    )
    def sc_embed(table_hbm, idx_hbm, dur_hbm, pos_hbm, out_hbm,
                 idx_v, dur_v, rows_v, pos_v, sem):
        wid = lax.axis_index("s") * NUM_CORES + lax.axis_index("c")
        base = wid * tok_per_w
        pltpu.sync_copy(pos_hbm, pos_v)

        @pl.loop(0, n_chunks)
        def _chunk(c):
            tok = base + c * CHUNK
            crow = wid * (tok_per_w // GATHER_W) + c * GATHERS
            pltpu.sync_copy(idx_hbm.at[pl.ds(crow, GATHERS)], idx_v)
            pltpu.sync_copy(dur_hbm.at[pl.ds(tok, CHUNK)], dur_v)
            copies = [
                pltpu.async_copy(
                    table_hbm.at[idx_v.at[j]],
                    rows_v.at[pl.ds(j * GATHER_W, GATHER_W)],
                    sem,
                )
                for j in range(GATHERS)
            ]
            for cp in copies:
                cp.wait()
            s0 = (c * CHUNK) % MAX_POS

            @pl.loop(0, CHUNK)
            def _tok(t):
                durb = plsc.load_gather(dur_v, [lax.broadcast(t, (LANES,))])
                row = s0 + t
                for d in range(EMBED_DIM // LANES):
                    sl = pl.ds(d * LANES, LANES)
                    rows_v[t, sl] = rows_v[t, sl] + pos_v[row, sl] + durb

            pltpu.sync_copy(rows_v, out_hbm.at[pl.ds(tok, CHUNK)])

    return sc_embed


def kernel(x_in, table):
    batch, seq, _ = x_in.shape
    n_tokens = batch * seq
    notes = x_in[:, :, 0].reshape(n_tokens // GATHER_W, GATHER_W)
    dur = x_in[:, :, 1].astype(jnp.float32).reshape(n_tokens)
    pos_ext = jnp.asarray(_pos_enc_ext(192 + CHUNK))
    out = _make_sc_embed(n_tokens)(table, notes, dur, pos_ext)
    return out.reshape(batch, seq, EMBED_DIM)


# speedup vs baseline: 4.0301x; 1.9318x over previous
import dataclasses
import functools

import jax
import jax.numpy as jnp
import numpy as np
from jax import lax
from jax.experimental import pallas as pl
from jax.experimental.pallas import tpu as pltpu
from jax.experimental.pallas import tpu_sc as plsc

MAX_POS = 200
EMBED_DIM = 64
LANES = 16

NUM_CORES = 2
NUM_SUBCORES = 16
NUM_WORKERS = NUM_CORES * NUM_SUBCORES

CHUNK = 256
GATHER_W = 128
GATHERS = CHUNK // GATHER_W
NBUF = 4
EXT_ROWS = 192 + CHUNK


def _pos_enc_ext(num_rows: int) -> np.ndarray:
    pos = np.arange(MAX_POS)[:, np.newaxis]
    i = np.arange(EMBED_DIM)[np.newaxis, :]
    angle_rates = 1 / np.power(10000, 2 * (i // 2) / np.float32(EMBED_DIM))
    angle_rads = pos * angle_rates
    angle_rads[:, 0::2] = np.sin(angle_rads[:, 0::2])
    angle_rads[:, 1::2] = np.cos(angle_rads[:, 1::2])
    pe = angle_rads.astype(np.float32)
    return pe[np.arange(num_rows) % MAX_POS]


def _make_sc_embed(n_tokens: int):
    tok_per_w = n_tokens // NUM_WORKERS
    n_chunks = tok_per_w // CHUNK
    assert tok_per_w % CHUNK == 0 and n_chunks % NBUF == 0
    mesh = plsc.VectorSubcoreMesh(core_axis_name="c", subcore_axis_name="s")
    cp = pltpu.CompilerParams()
    if "needs_layout_passes" in pltpu.CompilerParams.__dataclass_fields__:
        cp = dataclasses.replace(cp, needs_layout_passes=False)
    if "use_tc_tiling_on_sc" in pltpu.CompilerParams.__dataclass_fields__:
        cp = dataclasses.replace(cp, use_tc_tiling_on_sc=False)

    @functools.partial(
        pl.kernel,
        out_type=jax.ShapeDtypeStruct((n_tokens, EMBED_DIM), jnp.float32),
        mesh=mesh,
        compiler_params=cp,
        scratch_types=[
            pltpu.VMEM((NBUF, GATHERS, GATHER_W), jnp.int32),
            pltpu.VMEM((NBUF, CHUNK), jnp.float32),
            pltpu.VMEM((NBUF, CHUNK, EMBED_DIM), jnp.float32),
            pltpu.VMEM((EXT_ROWS, EMBED_DIM), jnp.float32),
            pltpu.SemaphoreType.DMA((NBUF,)),
            pltpu.SemaphoreType.DMA((NBUF,)),
            pltpu.SemaphoreType.DMA((NBUF,)),
        ],
    )
    def sc_embed(table_hbm, idx_hbm, dur_hbm, pos_hbm, out_hbm,
                 idx_v, dur_v, rows_v, pos_v, sem_in, sem_g, sem_out):
        wid = lax.axis_index("s") * NUM_CORES + lax.axis_index("c")
        base = wid * tok_per_w
        base_row = wid * (tok_per_w // GATHER_W)
        pltpu.sync_copy(pos_hbm, pos_v)

        def issue_in(c, b):
            pltpu.async_copy(
                idx_hbm.at[pl.ds(base_row + c * GATHERS, GATHERS)],
                idx_v.at[b], sem_in.at[b])
            pltpu.async_copy(
                dur_hbm.at[pl.ds(base + c * CHUNK, CHUNK)],
                dur_v.at[b], sem_in.at[b])

        def wait_in(b):
            pltpu.make_async_copy(
                idx_hbm.at[pl.ds(0, GATHERS)], idx_v.at[b], sem_in.at[b]).wait()
            pltpu.make_async_copy(
                dur_hbm.at[pl.ds(0, CHUNK)], dur_v.at[b], sem_in.at[b]).wait()

        def issue_gather(b):
            for j in range(GATHERS):
                pltpu.async_copy(
                    table_hbm.at[idx_v.at[b, j]],
                    rows_v.at[b].at[pl.ds(j * GATHER_W, GATHER_W)],
                    sem_g.at[b])

        def wait_gather(b):
            for j in range(GATHERS):
                pltpu.make_async_copy(
                    table_hbm.at[idx_v.at[b, j]],
                    rows_v.at[b].at[pl.ds(j * GATHER_W, GATHER_W)],
                    sem_g.at[b]).wait()

        def issue_out(c, b):
            pltpu.async_copy(
                rows_v.at[b], out_hbm.at[pl.ds(base + c * CHUNK, CHUNK)],
                sem_out.at[b])

        def wait_out(b):
            pltpu.make_async_copy(
                rows_v.at[b], out_hbm.at[pl.ds(0, CHUNK)], sem_out.at[b]).wait()

        issue_in(0, 0)
        issue_in(1, 1)
        wait_in(0)
        issue_gather(0)

        @pl.loop(0, n_chunks, step=NBUF)
        def _ring(cc):
            for b in range(NBUF):
                c = cc + b
                b1, b2, b3 = (b + 1) % NBUF, (b + 2) % NBUF, (b + 1) % NBUF

                @pl.when(c + 1 < n_chunks)
                def _():
                    wait_in(b1)

                @pl.when(c >= NBUF - 1)
                def _():
                    wait_out(b3)

                @pl.when(c + 1 < n_chunks)
                def _():
                    issue_gather(b1)

                @pl.when(c + 2 < n_chunks)
                def _():
                    issue_in(c + 2, b2)

                wait_gather(b)
                s0 = (c * CHUNK) % MAX_POS

                @plsc.parallel_loop(0, CHUNK, 1, unroll=4)
                def _tok(t):
                    durb = plsc.load_gather(
                        dur_v.at[b], [lax.broadcast(t, (LANES,))])
                    row = s0 + t
                    for d in range(EMBED_DIM // LANES):
                        sl = pl.ds(d * LANES, LANES)
                        rows_v[b, t, sl] = (
                            rows_v[b, t, sl] + pos_v[row, sl] + durb)

                issue_out(c, b)

        for k in range(n_chunks - NBUF + 1, n_chunks):
            wait_out(k % NBUF)

    return sc_embed


def kernel(x_in, table):
    batch, seq, _ = x_in.shape
    n_tokens = batch * seq
    notes = x_in[:, :, 0].reshape(n_tokens // GATHER_W, GATHER_W)
    dur = x_in[:, :, 1].astype(jnp.float32).reshape(n_tokens)
    pos_ext = jnp.asarray(_pos_enc_ext(EXT_ROWS))
    out = _make_sc_embed(n_tokens)(table, notes, dur, pos_ext)
    return out.reshape(batch, seq, EMBED_DIM)
